# baseline (speedup 1.0000x reference)
import functools

import jax
import jax.numpy as jnp
from jax.experimental import pallas as pl
from jax.experimental.pallas import tpu as pltpu
from jax.experimental.pallas import tpu_sc as plsc

_VOCAB = 1000000
_DIM = 128
_BATCH = 4096
_HIST = 200

_BB = 64
_TILES = 32
_GW = 128
_CHUNKS = (512,) * 8
_ROWS_PER_DMA = 1
_GR = _GW * _ROWS_PER_DMA
_NBUF = 5
_LAG = 2


def _gather(table, ids3, gpt):
    cids = _TILES * gpt * _GR
    mesh = plsc.VectorSubcoreMesh(core_axis_name="core",
                                  subcore_axis_name="subcore")

    @functools.partial(
        pl.kernel,
        out_type=jax.ShapeDtypeStruct((cids, _DIM), jnp.float32),
        mesh=mesh,
        scratch_types=[
            pltpu.VMEM((gpt, _GR), jnp.int32),
            pltpu.VMEM((_NBUF, _GR, _DIM), jnp.float32),
            pltpu.SemaphoreType.DMA((_NBUF,)),
            pltpu.SemaphoreType.DMA((_NBUF,)),
        ],
    )
    def sc_kernel(table_hbm, ids_hbm, out_hbm, idx_v, buf, gsem, osem):
        wid = (jax.lax.axis_index("subcore") * 2
               + jax.lax.axis_index("core"))
        pltpu.sync_copy(ids_hbm.at[wid], idx_v)
        base = wid * (gpt * _GR)

        for j in range(gpt + _LAG):
            b = j % _NBUF
            if j < gpt:
                if j >= _NBUF:
                    pltpu.make_async_copy(
                        buf.at[b], out_hbm.at[pl.ds(base, _GR)],
                        osem.at[b],
                    ).wait()
                pltpu.async_copy(
                    table_hbm.at[idx_v.at[j]], buf.at[b], gsem.at[b])
            if j >= _LAG:
                jj = j - _LAG
                bb = jj % _NBUF
                pltpu.make_async_copy(
                    table_hbm.at[idx_v.at[0]], buf.at[bb], gsem.at[bb],
                ).wait()
                pltpu.async_copy(
                    buf.at[bb],
                    out_hbm.at[pl.ds(base + jj * _GR, _GR)],
                    osem.at[bb],
                )
        for j in range(gpt - _NBUF, gpt):
            b = j % _NBUF
            pltpu.make_async_copy(
                buf.at[b], out_hbm.at[pl.ds(base, _GR)],
                osem.at[b],
            ).wait()

    return sc_kernel(table, ids3)


def _mlp_body(e_ref, w1t_ref, b1_ref, w2t_ref, b2_ref, s_ref, o_ref):
    h = jnp.dot(e_ref[...].astype(jnp.bfloat16), w1t_ref[...],
                preferred_element_type=jnp.float32) + b1_ref[...]
    h = jnp.maximum(h, 0.0)
    hm = jnp.dot(s_ref[...], h.astype(jnp.bfloat16),
                 preferred_element_type=jnp.float32)
    o_ref[...] = jnp.dot(hm, w2t_ref[...],
                         preferred_element_type=jnp.float32) + b2_ref[...]


def _mlp(embeds, w1t, b1, w2t, b2, seg, cb):
    grid = cb // _BB
    return pl.pallas_call(
        _mlp_body,
        grid=(grid,),
        in_specs=[
            pl.BlockSpec((_BB * _HIST, _DIM), lambda i: (i, 0)),
            pl.BlockSpec((_DIM, _DIM), lambda i: (0, 0)),
            pl.BlockSpec((1, _DIM), lambda i: (0, 0)),
            pl.BlockSpec((_DIM, _DIM), lambda i: (0, 0)),
            pl.BlockSpec((1, _DIM), lambda i: (0, 0)),
            pl.BlockSpec((_BB, _BB * _HIST), lambda i: (0, 0)),
        ],
        out_specs=pl.BlockSpec((_BB, _DIM), lambda i: (i, 0)),
        out_shape=jax.ShapeDtypeStruct((cb, _DIM), jnp.float32),
    )(embeds, w1t, b1, w2t, b2, seg)


def kernel(input_ids, table, W1, b1, W2, b2):
    flat_ids = input_ids.reshape(-1).astype(jnp.int32)
    col = jax.lax.broadcasted_iota(jnp.int32, (_BB, _BB * _HIST), 1)
    row = jax.lax.broadcasted_iota(jnp.int32, (_BB, _BB * _HIST), 0)
    seg = jnp.where(col // _HIST == row, jnp.float32(1.0),
                    jnp.float32(0.0)).astype(jnp.bfloat16)
    w1t = W1.T.astype(jnp.bfloat16)
    w2t = W2.T * jnp.float32(1.0 / _HIST)
    b1r, b2r = b1.reshape(1, _DIM), b2.reshape(1, _DIM)
    outs = []
    start = 0
    for cb in _CHUNKS:
        cids = cb * _HIST
        gpt = cids // (_TILES * _GR)
        ids3 = jax.lax.dynamic_slice_in_dim(flat_ids, start * _HIST, cids)
        ids3 = ids3.reshape(_TILES, gpt, _GR)
        emb_c = _gather(table, ids3, gpt)
        outs.append(_mlp(emb_c, w1t, b1r, w2t, b2r, seg, cb))
        start += cb
    return jnp.concatenate(outs, axis=0)


# speedup vs baseline: 1.0028x; 1.0028x over previous
import functools

import jax
import jax.numpy as jnp
from jax.experimental import pallas as pl
from jax.experimental.pallas import tpu as pltpu
from jax.experimental.pallas import tpu_sc as plsc

_VOCAB = 1000000
_DIM = 128
_BATCH = 4096
_HIST = 200

_BB = 64
_TILES = 32
_GW = 128
_CHUNKS = (2048, 2048)
_ROWS_PER_DMA = 1
_GR = _GW * _ROWS_PER_DMA
_NBUF = 5
_LAG = 2


def _gather(table, ids3, gpt):
    cids = _TILES * gpt * _GR
    mesh = plsc.VectorSubcoreMesh(core_axis_name="core",
                                  subcore_axis_name="subcore")

    @functools.partial(
        pl.kernel,
        out_type=jax.ShapeDtypeStruct((cids, _DIM), jnp.float32),
        mesh=mesh,
        scratch_types=[
            pltpu.VMEM((gpt, _GR), jnp.int32),
            pltpu.VMEM((_NBUF, _GR, _DIM), jnp.float32),
            pltpu.SemaphoreType.DMA((_NBUF,)),
            pltpu.SemaphoreType.DMA((_NBUF,)),
        ],
    )
    def sc_kernel(table_hbm, ids_hbm, out_hbm, idx_v, buf, gsem, osem):
        wid = (jax.lax.axis_index("subcore") * 2
               + jax.lax.axis_index("core"))
        pltpu.sync_copy(ids_hbm.at[wid], idx_v)
        base = wid * (gpt * _GR)

        for j in range(gpt + _LAG):
            b = j % _NBUF
            if j < gpt:
                if j >= _NBUF:
                    pltpu.make_async_copy(
                        buf.at[b], out_hbm.at[pl.ds(base, _GR)],
                        osem.at[b],
                    ).wait()
                pltpu.async_copy(
                    table_hbm.at[idx_v.at[j]], buf.at[b], gsem.at[b])
            if j >= _LAG:
                jj = j - _LAG
                bb = jj % _NBUF
                pltpu.make_async_copy(
                    table_hbm.at[idx_v.at[0]], buf.at[bb], gsem.at[bb],
                ).wait()
                pltpu.async_copy(
                    buf.at[bb],
                    out_hbm.at[pl.ds(base + jj * _GR, _GR)],
                    osem.at[bb],
                )
        for j in range(gpt - _NBUF, gpt):
            b = j % _NBUF
            pltpu.make_async_copy(
                buf.at[b], out_hbm.at[pl.ds(base, _GR)],
                osem.at[b],
            ).wait()

    return sc_kernel(table, ids3)


def _mlp_body(e_ref, w1t_ref, b1_ref, w2t_ref, b2_ref, s_ref, o_ref):
    h = jnp.dot(e_ref[...].astype(jnp.bfloat16), w1t_ref[...],
                preferred_element_type=jnp.float32) + b1_ref[...]
    h = jnp.maximum(h, 0.0)
    hm = jnp.dot(s_ref[...], h.astype(jnp.bfloat16),
                 preferred_element_type=jnp.float32)
    o_ref[...] = jnp.dot(hm, w2t_ref[...],
                         preferred_element_type=jnp.float32) + b2_ref[...]


def _mlp(embeds, w1t, b1, w2t, b2, seg, cb):
    grid = cb // _BB
    return pl.pallas_call(
        _mlp_body,
        grid=(grid,),
        in_specs=[
            pl.BlockSpec((_BB * _HIST, _DIM), lambda i: (i, 0)),
            pl.BlockSpec((_DIM, _DIM), lambda i: (0, 0)),
            pl.BlockSpec((1, _DIM), lambda i: (0, 0)),
            pl.BlockSpec((_DIM, _DIM), lambda i: (0, 0)),
            pl.BlockSpec((1, _DIM), lambda i: (0, 0)),
            pl.BlockSpec((_BB, _BB * _HIST), lambda i: (0, 0)),
        ],
        out_specs=pl.BlockSpec((_BB, _DIM), lambda i: (i, 0)),
        out_shape=jax.ShapeDtypeStruct((cb, _DIM), jnp.float32),
    )(embeds, w1t, b1, w2t, b2, seg)


def kernel(input_ids, table, W1, b1, W2, b2):
    flat_ids = input_ids.reshape(-1).astype(jnp.int32)
    col = jax.lax.broadcasted_iota(jnp.int32, (_BB, _BB * _HIST), 1)
    row = jax.lax.broadcasted_iota(jnp.int32, (_BB, _BB * _HIST), 0)
    seg = jnp.where(col // _HIST == row, jnp.float32(1.0),
                    jnp.float32(0.0)).astype(jnp.bfloat16)
    w1t = W1.T.astype(jnp.bfloat16)
    w2t = W2.T * jnp.float32(1.0 / _HIST)
    b1r, b2r = b1.reshape(1, _DIM), b2.reshape(1, _DIM)
    outs = []
    start = 0
    for cb in _CHUNKS:
        cids = cb * _HIST
        gpt = cids // (_TILES * _GR)
        ids3 = jax.lax.dynamic_slice_in_dim(flat_ids, start * _HIST, cids)
        ids3 = ids3.reshape(_TILES, gpt, _GR)
        emb_c = _gather(table, ids3, gpt)
        outs.append(_mlp(emb_c, w1t, b1r, w2t, b2r, seg, cb))
        start += cb
    return jnp.concatenate(outs, axis=0)


# speedup vs baseline: 1.0160x; 1.0131x over previous
import functools

import jax
import jax.numpy as jnp
from jax.experimental import pallas as pl
from jax.experimental.pallas import tpu as pltpu
from jax.experimental.pallas import tpu_sc as plsc

_VOCAB = 1000000
_DIM = 128
_BATCH = 4096
_HIST = 200

_BB = 64
_TILES = 32
_GW = 128
_CHUNKS = (1024, 1024, 1024, 1024)
_ROWS_PER_DMA = 1
_GR = _GW * _ROWS_PER_DMA
_NBUF = 5
_LAG = 2


def _gather(table, ids3, gpt):
    cids = _TILES * gpt * _GR
    mesh = plsc.VectorSubcoreMesh(core_axis_name="core",
                                  subcore_axis_name="subcore")

    @functools.partial(
        pl.kernel,
        out_type=jax.ShapeDtypeStruct((cids, _DIM), jnp.float32),
        mesh=mesh,
        scratch_types=[
            pltpu.VMEM((gpt, _GR), jnp.int32),
            pltpu.VMEM((_NBUF, _GR, _DIM), jnp.float32),
            pltpu.SemaphoreType.DMA((_NBUF,)),
            pltpu.SemaphoreType.DMA((_NBUF,)),
        ],
    )
    def sc_kernel(table_hbm, ids_hbm, out_hbm, idx_v, buf, gsem, osem):
        wid = (jax.lax.axis_index("subcore") * 2
               + jax.lax.axis_index("core"))
        pltpu.sync_copy(ids_hbm.at[wid], idx_v)
        base = wid * (gpt * _GR)

        for j in range(gpt + _LAG):
            b = j % _NBUF
            if j < gpt:
                if j >= _NBUF:
                    pltpu.make_async_copy(
                        buf.at[b], out_hbm.at[pl.ds(base, _GR)],
                        osem.at[b],
                    ).wait()
                pltpu.async_copy(
                    table_hbm.at[idx_v.at[j]], buf.at[b], gsem.at[b])
            if j >= _LAG:
                jj = j - _LAG
                bb = jj % _NBUF
                pltpu.make_async_copy(
                    table_hbm.at[idx_v.at[0]], buf.at[bb], gsem.at[bb],
                ).wait()
                pltpu.async_copy(
                    buf.at[bb],
                    out_hbm.at[pl.ds(base + jj * _GR, _GR)],
                    osem.at[bb],
                )
        for j in range(gpt - _NBUF, gpt):
            b = j % _NBUF
            pltpu.make_async_copy(
                buf.at[b], out_hbm.at[pl.ds(base, _GR)],
                osem.at[b],
            ).wait()

    return sc_kernel(table, ids3)


def _mlp_body(e_ref, w1t_ref, b1_ref, w2t_ref, b2_ref, s_ref, o_ref):
    h = jnp.dot(e_ref[...].astype(jnp.bfloat16), w1t_ref[...],
                preferred_element_type=jnp.float32) + b1_ref[...]
    h = jnp.maximum(h, 0.0)
    hm = jnp.dot(s_ref[...], h.astype(jnp.bfloat16),
                 preferred_element_type=jnp.float32)
    o_ref[...] = jnp.dot(hm, w2t_ref[...],
                         preferred_element_type=jnp.float32) + b2_ref[...]


def _mlp(embeds, w1t, b1, w2t, b2, seg, cb):
    grid = cb // _BB
    return pl.pallas_call(
        _mlp_body,
        grid=(grid,),
        in_specs=[
            pl.BlockSpec((_BB * _HIST, _DIM), lambda i: (i, 0)),
            pl.BlockSpec((_DIM, _DIM), lambda i: (0, 0)),
            pl.BlockSpec((1, _DIM), lambda i: (0, 0)),
            pl.BlockSpec((_DIM, _DIM), lambda i: (0, 0)),
            pl.BlockSpec((1, _DIM), lambda i: (0, 0)),
            pl.BlockSpec((_BB, _BB * _HIST), lambda i: (0, 0)),
        ],
        out_specs=pl.BlockSpec((_BB, _DIM), lambda i: (i, 0)),
        out_shape=jax.ShapeDtypeStruct((cb, _DIM), jnp.float32),
    )(embeds, w1t, b1, w2t, b2, seg)


def kernel(input_ids, table, W1, b1, W2, b2):
    flat_ids = input_ids.reshape(-1).astype(jnp.int32)
    col = jax.lax.broadcasted_iota(jnp.int32, (_BB, _BB * _HIST), 1)
    row = jax.lax.broadcasted_iota(jnp.int32, (_BB, _BB * _HIST), 0)
    seg = jnp.where(col // _HIST == row, jnp.float32(1.0),
                    jnp.float32(0.0)).astype(jnp.bfloat16)
    w1t = W1.T.astype(jnp.bfloat16)
    w2t = W2.T * jnp.float32(1.0 / _HIST)
    b1r, b2r = b1.reshape(1, _DIM), b2.reshape(1, _DIM)
    outs = []
    start = 0
    for cb in _CHUNKS:
        cids = cb * _HIST
        gpt = cids // (_TILES * _GR)
        ids3 = jax.lax.dynamic_slice_in_dim(flat_ids, start * _HIST, cids)
        ids3 = ids3.reshape(_TILES, gpt, _GR)
        emb_c = _gather(table, ids3, gpt)
        outs.append(_mlp(emb_c, w1t, b1r, w2t, b2r, seg, cb))
        start += cb
    return jnp.concatenate(outs, axis=0)
